# baseline (speedup 1.0000x reference)
import functools

import jax
import jax.numpy as jnp
import numpy as np
from jax import lax
from jax.experimental import pallas as pl
from jax.experimental.pallas import tpu as pltpu
from jax.experimental.pallas import tpu_sc as plsc

N_NODES = 10000
N_EDGES = 320000
D = 128
D_EDGE = 16
D_GLOBAL = 16
N_GRAPHS = 8
DC = D + 16
NC, NS = 2, 16
NW = NC * NS
EPW = N_EDGES // NW
CH = 128
NROW = N_EDGES // CH
N_PAD = 10240
ROWS_PER_SUB = N_PAD // NS


def _pack_bf16(a, b):
    a = lax.bitcast_convert_type(a, jnp.int32)
    b = lax.bitcast_convert_type(b, jnp.int32)
    a = a + 0x7FFF + jnp.bitwise_and(lax.shift_right_logical(a, 16), 1)
    b = b + 0x7FFF + jnp.bitwise_and(lax.shift_right_logical(b, 16), 1)
    lo = lax.shift_right_logical(a, 16)
    hi = jnp.bitwise_and(b, jnp.int32(-65536))
    return jnp.bitwise_or(hi, lo)


def _mm_kernel(a_ref, b_ref, o_ref):
    m = jnp.dot(a_ref[...], b_ref[...], preferred_element_type=jnp.float32)
    o_ref[...] = _pack_bf16(m[:, :D // 2], m[:, D // 2:])


def _edge_pre_kernel(eat_ref, w_ref, b_ref, o_ref):
    m = lax.dot_general(
        eat_ref[...], w_ref[...], (((0,), (0,)), ((), ())),
        preferred_element_type=jnp.float32) + b_ref[...]
    m3 = m.reshape(m.shape[0] // CH, CH, D)
    w3 = _pack_bf16(m3[:, :CH // 2, :], m3[:, CH // 2:, :])
    o_ref[...] = w3.reshape(m.shape[0] // 2, D)


def _sc_body(xw1_hbm, eb_hbm, row_hbm, col_hbm, out_hbm,
             acc_sh, row_v, col_v, e_v, g_v, h_v, rs0, rs1, gsem, esem, ssem):
    cid = lax.axis_index("c")
    sid = lax.axis_index("s")
    wid = sid * NC + cid
    rsem = (rs0, rs1)

    zero16 = jnp.zeros((16,), jnp.float32)
    one16 = jnp.ones((16,), jnp.float32)

    @plsc.parallel_loop(0, CH, 1, unroll=4)
    def zfill(i):
        for j in range(DC // 16):
            h_v[i, pl.ds(j * 16, 16)] = zero16

    def zcopy(i, _):
        pltpu.sync_copy(h_v, acc_sh.at[pl.ds(sid * ROWS_PER_SUB + i * CH, CH), :])
        return 0
    lax.fori_loop(0, ROWS_PER_SUB // CH, zcopy, 0)

    @plsc.parallel_loop(0, CH, 1, unroll=4)
    def onefill(i):
        h_v[i, pl.ds(D, 16)] = one16
    plsc.subcore_barrier()

    nrows = jnp.where(wid < 4, NROW // NW + 1, NROW // NW)
    rstart = NROW // NW * wid + jnp.minimum(wid, 4)

    def idx_issue(r, b):
        pltpu.async_copy(row_hbm.at[pl.ds(r, 1), :], row_v.at[pl.ds(b, 1), :],
                         rsem[b])
        pltpu.async_copy(col_hbm.at[pl.ds(r, 1), :], col_v.at[pl.ds(b, 1), :],
                         rsem[b])

    def data_issue(r, bb):
        pltpu.async_copy(eb_hbm.at[pl.ds(r * (CH // 2), CH // 2), :], e_v, esem)
        pltpu.async_copy(xw1_hbm.at[row_v.at[bb]], g_v, gsem)

    idx_issue(rstart, 0)
    pltpu.make_async_copy(row_hbm.at[pl.ds(0, 1), :],
                          row_v.at[pl.ds(0, 1), :], rsem[0]).wait()
    pltpu.make_async_copy(col_hbm.at[pl.ds(0, 1), :],
                          col_v.at[pl.ds(0, 1), :], rsem[0]).wait()
    data_issue(rstart, 0)

    @pl.when(1 < nrows)
    def _():
        idx_issue(rstart + 1, 1)

    mask_hi = jnp.full((16,), -65536, jnp.int32)

    def chunk(c, _):
        r = rstart + c
        b = lax.rem(c, 2)
        pltpu.make_async_copy(eb_hbm.at[pl.ds(0, CH // 2), :], e_v, esem).wait()
        pltpu.make_async_copy(xw1_hbm.at[row_v.at[0]], g_v, gsem).wait()

        @plsc.parallel_loop(0, CH // 2, 1, unroll=4)
        def fuse(q):
            for j in range(D // 32):
                sl = pl.ds(j * 16, 16)
                sh = pl.ds(D // 2 + j * 16, 16)
                ew = e_v[q, sl]
                ew2 = e_v[q, sh]
                gw = g_v[q, sl]
                gw2 = g_v[CH // 2 + q, sl]
                glo = plsc.bitcast(lax.shift_left(gw, 16), jnp.float32)
                ghi = plsc.bitcast(jnp.bitwise_and(gw, mask_hi), jnp.float32)
                glo2 = plsc.bitcast(lax.shift_left(gw2, 16), jnp.float32)
                ghi2 = plsc.bitcast(jnp.bitwise_and(gw2, mask_hi), jnp.float32)
                elo = plsc.bitcast(lax.shift_left(ew, 16), jnp.float32)
                ehi = plsc.bitcast(jnp.bitwise_and(ew, mask_hi), jnp.float32)
                elo2 = plsc.bitcast(lax.shift_left(ew2, 16), jnp.float32)
                ehi2 = plsc.bitcast(jnp.bitwise_and(ew2, mask_hi), jnp.float32)
                h_v[q, sl] = jnp.maximum(elo + glo, 0.0)
                h_v[q, sh] = jnp.maximum(elo2 + ghi, 0.0)
                h_v[CH // 2 + q, sl] = jnp.maximum(ehi + glo2, 0.0)
                h_v[CH // 2 + q, sh] = jnp.maximum(ehi2 + ghi2, 0.0)

        @pl.when(c + 1 < nrows)
        def _():
            for bb in range(2):
                @pl.when(b == bb)
                def _():
                    b1 = 1 - bb
                    pltpu.make_async_copy(row_hbm.at[pl.ds(0, 1), :],
                                          row_v.at[pl.ds(b1, 1), :],
                                          rsem[b1]).wait()
                    pltpu.make_async_copy(col_hbm.at[pl.ds(0, 1), :],
                                          col_v.at[pl.ds(b1, 1), :],
                                          rsem[b1]).wait()
                    data_issue(r + 1, b1)

        for bb in range(2):
            @pl.when(b == bb)
            def _():
                pltpu.async_copy(h_v, acc_sh.at[col_v.at[bb]], ssem,
                                 add=True).wait()

        @pl.when(c + 2 < nrows)
        def _():
            for bb in range(2):
                @pl.when(b == bb)
                def _():
                    idx_issue(r + 2, bb)
        return 0
    lax.fori_loop(0, nrows, chunk, 0)

    plsc.subcore_barrier()
    pltpu.sync_copy(
        acc_sh.at[pl.ds(sid * ROWS_PER_SUB, ROWS_PER_SUB), :],
        out_hbm.at[cid, pl.ds(sid * ROWS_PER_SUB, ROWS_PER_SUB), :],
    )


def _node_mlp_kernel(x_ref, s_ref, batch_ref, u_ref, w2_ref, b2_ref,
                     w3_ref, b3_ref, w4_ref, b4_ref, o_ref):
    s = s_ref[0, :, :D] + s_ref[1, :, :D]
    c = (s_ref[0, :, D:D + 1] + s_ref[1, :, D:D + 1])
    inv = 1.0 / jnp.maximum(c, 1.0)
    mean = (jnp.dot(s, w2_ref[...], preferred_element_type=jnp.float32)
            + c * b2_ref[...]) * inv
    b = batch_ref[0, 0, :]
    oh = (b[:, None] == lax.broadcasted_iota(jnp.int32, (b.shape[0], N_GRAPHS), 1))
    uc = jnp.dot(u_ref[...], w3_ref[D + D:, :], preferred_element_type=jnp.float32)
    t = (jnp.dot(x_ref[...], w3_ref[:D, :], preferred_element_type=jnp.float32)
         + jnp.dot(mean, w3_ref[D:D + D, :], preferred_element_type=jnp.float32)
         + jnp.dot(oh.astype(jnp.float32), uc, preferred_element_type=jnp.float32)
         + b3_ref[...])
    o_ref[...] = (jnp.dot(jnp.maximum(t, 0.0), w4_ref[...],
                          preferred_element_type=jnp.float32) + b4_ref[...])


def kernel(x, edge_index, edge_attr, u, batch, W1, b1, W2, b2, W3, b3, W4, b4):
    row = edge_index[0].astype(jnp.int32)
    col = edge_index[1].astype(jnp.int32)
    W1a = W1[:D]
    W1b = W1[D:]

    xw1 = pl.pallas_call(
        _mm_kernel,
        grid=(5,),
        in_specs=[
            pl.BlockSpec((N_NODES // 5, D), lambda i: (i, 0)),
            pl.BlockSpec((D, D), lambda i: (0, 0)),
        ],
        out_specs=pl.BlockSpec((N_NODES // 5, D // 2), lambda i: (i, 0)),
        out_shape=jax.ShapeDtypeStruct((N_NODES, D // 2), jnp.int32),
    )(x, W1a)

    EB_BLK = 6400
    eb = pl.pallas_call(
        _edge_pre_kernel,
        grid=(N_EDGES // EB_BLK,),
        in_specs=[
            pl.BlockSpec((D_EDGE, EB_BLK), lambda i: (0, i)),
            pl.BlockSpec((D_EDGE, D), lambda i: (0, 0)),
            pl.BlockSpec((D,), lambda i: (0,)),
        ],
        out_specs=pl.BlockSpec((EB_BLK // 2, D), lambda i: (i, 0)),
        out_shape=jax.ShapeDtypeStruct((N_EDGES // 2, D), jnp.int32),
    )(edge_attr.T, W1b, b1)

    mesh = plsc.VectorSubcoreMesh(core_axis_name="c", subcore_axis_name="s")
    sc = functools.partial(
        pl.kernel,
        mesh=mesh,
        out_type=jax.ShapeDtypeStruct((NC, N_PAD, DC), jnp.float32),
        scratch_types=[
            pltpu.VMEM_SHARED((N_PAD, DC), jnp.float32),
            pltpu.VMEM((2, CH), jnp.int32),
            pltpu.VMEM((2, CH), jnp.int32),
            pltpu.VMEM((CH // 2, D), jnp.int32),
            pltpu.VMEM((CH, D // 2), jnp.int32),
            pltpu.VMEM((CH, DC), jnp.float32),
            pltpu.SemaphoreType.DMA,
            pltpu.SemaphoreType.DMA,
            pltpu.SemaphoreType.DMA,
            pltpu.SemaphoreType.DMA,
            pltpu.SemaphoreType.DMA,
        ],
        compiler_params=pltpu.CompilerParams(use_tc_tiling_on_sc=False,
                                             needs_layout_passes=False),
    )(_sc_body)
    s01 = sc(xw1, eb, row.reshape(NROW, CH), col.reshape(NROW, CH))

    R = 1000
    batch3 = batch.astype(jnp.int32).reshape(N_NODES // R, 1, R)
    out = pl.pallas_call(
        _node_mlp_kernel,
        grid=(N_NODES // R,),
        in_specs=[
            pl.BlockSpec((R, D), lambda i: (i, 0)),
            pl.BlockSpec((NC, R, DC), lambda i: (0, i, 0)),
            pl.BlockSpec((1, 1, R), lambda i: (i, 0, 0)),
            pl.BlockSpec((N_GRAPHS, D_GLOBAL), lambda i: (0, 0)),
            pl.BlockSpec((D, D), lambda i: (0, 0)),
            pl.BlockSpec((D,), lambda i: (0,)),
            pl.BlockSpec((D + D + D_GLOBAL, D), lambda i: (0, 0)),
            pl.BlockSpec((D,), lambda i: (0,)),
            pl.BlockSpec((D, D), lambda i: (0, 0)),
            pl.BlockSpec((D,), lambda i: (0,)),
        ],
        out_specs=pl.BlockSpec((R, D), lambda i: (i, 0)),
        out_shape=jax.ShapeDtypeStruct((N_NODES, D), jnp.float32),
    )(x, s01, batch3, u, W2, b2, W3, b3, W4, b4)
    return out
